# baseline (speedup 1.0000x reference)
import functools

import jax
import jax.numpy as jnp
from jax import lax
from jax.experimental import pallas as pl
from jax.experimental.pallas import tpu as pltpu
from jax.experimental.pallas import tpu_sc as plsc

CHUNK = 128
NBUF = 4
LANES = 16


@functools.lru_cache(maxsize=None)
def _make_lookup(n_rows, dim, seq):
    info = plsc.get_sparse_core_info()
    nc, ns = info.num_cores, info.num_subcores
    nw = nc * ns
    n_chunks = n_rows // CHUNK
    assert n_chunks % nw == 0, (n_rows, nw, CHUNK)
    chunks_per_w = n_chunks // nw
    assert chunks_per_w % NBUF == 0, (chunks_per_w, NBUF)
    n_rounds = chunks_per_w // NBUF
    mesh = plsc.VectorSubcoreMesh(core_axis_name="c", subcore_axis_name="s")
    bgroups = n_rows // (CHUNK * seq)

    @functools.partial(
        pl.kernel,
        mesh=mesh,
        out_type=jax.ShapeDtypeStruct((seq * (dim // 8) * bgroups * 8, CHUNK),
                                      jnp.float32),
        scratch_types=(
            [pltpu.VMEM((chunks_per_w, CHUNK), jnp.int32)]
            + [pltpu.VMEM((CHUNK, dim), jnp.float32)] * NBUF
            + [pltpu.VMEM((dim, CHUNK), jnp.float32)] * NBUF
            + [pltpu.SemaphoreType.DMA] * (2 * NBUF)
        ),
        compiler_params=pltpu.CompilerParams(
            use_tc_tiling_on_sc=False, needs_layout_passes=False),
    )
    def lookup(emb_hbm, idx_hbm, out_hbm, idx_v, *bufs):
        rows_v = bufs[:NBUF]
        tbuf_v = bufs[NBUF:2 * NBUF]
        gsem = bufs[2 * NBUF:3 * NBUF]
        ssem = bufs[3 * NBUF:]
        wid = lax.axis_index("s") * nc + lax.axis_index("c")
        base = wid * chunks_per_w
        pltpu.sync_copy(idx_hbm.at[pl.ds(base, chunks_per_w)], idx_v)

        row_ids = [lax.iota(jnp.int32, LANES) + lg * LANES
                   for lg in range(CHUNK // LANES)]

        def gather(cl, b):
            return pltpu.make_async_copy(
                emb_hbm.at[idx_v.at[cl]], rows_v[b], gsem[b])

        def out_copy(cl, b, hg):
            c = base + cl
            s = c // bgroups
            bg = lax.rem(c, bgroups)
            row0 = ((s * (dim // 8) + hg) * bgroups + bg) * 8
            return pltpu.make_async_copy(
                tbuf_v[b].at[pl.ds(hg * 8, 8)], out_hbm.at[pl.ds(row0, 8)],
                ssem[b])

        def transpose(b):
            @plsc.parallel_loop(0, dim, 1, unroll=16)
            def h_body(h):
                hcol = jnp.full((LANES,), 0, jnp.int32) + h
                for lg in range(CHUNK // LANES):
                    vals = plsc.load_gather(rows_v[b], [row_ids[lg], hcol])
                    tbuf_v[b][h, pl.ds(lg * LANES, LANES)] = vals

        def do_chunk(cl, b, drain_prev, fire_next):
            gather(cl, b).wait()
            if drain_prev:
                for hg in range(dim // 8):
                    out_copy(cl, b, hg).wait()
            transpose(b)
            for hg in range(dim // 8):
                out_copy(cl, b, hg).start()
            if fire_next:
                gather(cl + NBUF, b).start()

        for b in range(NBUF):
            gather(b, b).start()

        def round_body(r, carry):
            for b in range(NBUF):
                do_chunk(r * NBUF + b, b, drain_prev=True, fire_next=True)
            return carry

        for b in range(NBUF):
            do_chunk(b, b, drain_prev=False, fire_next=True)
        lax.fori_loop(1, n_rounds - 1, round_body, 0)
        c0 = (n_rounds - 1) * NBUF
        for b in range(NBUF):
            do_chunk(c0 + b, b, drain_prev=True, fire_next=False)
        for b in range(NBUF):
            for hg in range(dim // 8):
                out_copy(c0 + b, b, hg).wait()

    return lookup


def kernel(item_seqs, emb):
    bsz, seq = item_seqs.shape
    _, dim = emb.shape
    n_rows = bsz * seq
    idx2d = item_seqs.T.reshape(n_rows // CHUNK, CHUNK)
    out2 = _make_lookup(n_rows, dim, seq)(emb, idx2d)
    out5 = out2.reshape(seq, dim // 8, bsz // CHUNK, 8, CHUNK)
    return out5.transpose(2, 4, 0, 1, 3).reshape(bsz, seq, dim)


# speedup vs baseline: 1.7182x; 1.7182x over previous
import functools

import jax
import jax.numpy as jnp
from jax import lax
from jax.experimental import pallas as pl
from jax.experimental.pallas import tpu as pltpu
from jax.experimental.pallas import tpu_sc as plsc

CHUNK = 128
NBUF = 4
LANES = 16


@functools.lru_cache(maxsize=None)
def _make_lookup(n_rows, dim, seq):
    info = plsc.get_sparse_core_info()
    nc, ns = info.num_cores, info.num_subcores
    nw = nc * ns
    n_chunks = n_rows // CHUNK
    assert n_chunks % nw == 0, (n_rows, nw, CHUNK)
    chunks_per_w = n_chunks // nw
    assert chunks_per_w % NBUF == 0, (chunks_per_w, NBUF)
    n_rounds = chunks_per_w // NBUF
    mesh = plsc.VectorSubcoreMesh(core_axis_name="c", subcore_axis_name="s")
    bgroups = n_rows // (CHUNK * seq)

    @functools.partial(
        pl.kernel,
        mesh=mesh,
        out_type=jax.ShapeDtypeStruct((seq * (dim // 8) * bgroups * 8, CHUNK),
                                      jnp.float32),
        scratch_types=(
            [pltpu.VMEM((chunks_per_w, CHUNK), jnp.int32)]
            + [pltpu.VMEM((CHUNK, dim), jnp.float32)] * NBUF
            + [pltpu.VMEM((dim, CHUNK), jnp.float32)] * NBUF
            + [pltpu.SemaphoreType.DMA] * (2 * NBUF)
        ),
        compiler_params=pltpu.CompilerParams(
            use_tc_tiling_on_sc=False, needs_layout_passes=False),
    )
    def lookup(emb_hbm, idx_hbm, out_hbm, idx_v, *bufs):
        rows_v = bufs[:NBUF]
        tbuf_v = bufs[NBUF:2 * NBUF]
        gsem = bufs[2 * NBUF:3 * NBUF]
        ssem = bufs[3 * NBUF:]
        wid = lax.axis_index("s") * nc + lax.axis_index("c")
        base = wid * chunks_per_w
        pltpu.sync_copy(idx_hbm.at[pl.ds(base, chunks_per_w)], idx_v)

        row_ids = [lax.iota(jnp.int32, LANES) + lg * LANES
                   for lg in range(CHUNK // LANES)]

        def gather(cl, b):
            return pltpu.make_async_copy(
                emb_hbm.at[idx_v.at[cl]], rows_v[b], gsem[b])

        def out_copy(cl, b, hg):
            c = base + cl
            s = c // bgroups
            bg = lax.rem(c, bgroups)
            row0 = ((s * (dim // 8) + hg) * bgroups + bg) * 8
            return pltpu.make_async_copy(
                tbuf_v[b].at[pl.ds(hg * 8, 8)], out_hbm.at[pl.ds(row0, 8)],
                ssem[b])

        iota16 = lax.iota(jnp.int32, LANES)

        def transpose(b):
            @plsc.parallel_loop(0, dim, 1, unroll=8)
            def h_body(h):
                hvec = lax.bitwise_and(iota16 + h, jnp.int32(dim - 1))
                for lg in range(CHUNK // LANES):
                    vals = plsc.load_gather(rows_v[b], [row_ids[lg], hvec])
                    plsc.store_scatter(tbuf_v[b], [hvec, row_ids[lg]], vals)

        def do_chunk(cl, b, drain_prev, fire_next):
            gather(cl, b).wait()
            if drain_prev:
                for hg in range(dim // 8):
                    out_copy(cl, b, hg).wait()
            transpose(b)
            for hg in range(dim // 8):
                out_copy(cl, b, hg).start()
            if fire_next:
                gather(cl + NBUF, b).start()

        for b in range(NBUF):
            gather(b, b).start()

        def round_body(r, carry):
            for b in range(NBUF):
                do_chunk(r * NBUF + b, b, drain_prev=True, fire_next=True)
            return carry

        for b in range(NBUF):
            do_chunk(b, b, drain_prev=False, fire_next=True)
        lax.fori_loop(1, n_rounds - 1, round_body, 0)
        c0 = (n_rounds - 1) * NBUF
        for b in range(NBUF):
            do_chunk(c0 + b, b, drain_prev=True, fire_next=False)
        for b in range(NBUF):
            for hg in range(dim // 8):
                out_copy(c0 + b, b, hg).wait()

    return lookup


def kernel(item_seqs, emb):
    bsz, seq = item_seqs.shape
    _, dim = emb.shape
    n_rows = bsz * seq
    idx2d = item_seqs.T.reshape(n_rows // CHUNK, CHUNK)
    out2 = _make_lookup(n_rows, dim, seq)(emb, idx2d)
    out5 = out2.reshape(seq, dim // 8, bsz // CHUNK, 8, CHUNK)
    return out5.transpose(2, 4, 0, 1, 3).reshape(bsz, seq, dim)


# speedup vs baseline: 1.7226x; 1.0026x over previous
import functools

import jax
import jax.numpy as jnp
from jax import lax
from jax.experimental import pallas as pl
from jax.experimental.pallas import tpu as pltpu
from jax.experimental.pallas import tpu_sc as plsc

CHUNK = 128
NBUF = 4
LANES = 16


@functools.lru_cache(maxsize=None)
def _make_lookup(n_rows, dim, seq):
    info = plsc.get_sparse_core_info()
    nc, ns = info.num_cores, info.num_subcores
    nw = nc * ns
    n_chunks = n_rows // CHUNK
    assert n_chunks % nw == 0, (n_rows, nw, CHUNK)
    chunks_per_w = n_chunks // nw
    assert chunks_per_w % NBUF == 0, (chunks_per_w, NBUF)
    n_rounds = chunks_per_w // NBUF
    mesh = plsc.VectorSubcoreMesh(core_axis_name="c", subcore_axis_name="s")
    bgroups = n_rows // (CHUNK * seq)

    @functools.partial(
        pl.kernel,
        mesh=mesh,
        out_type=jax.ShapeDtypeStruct((seq * (dim // 8) * bgroups * 8, CHUNK),
                                      jnp.float32),
        scratch_types=(
            [pltpu.VMEM((chunks_per_w, CHUNK), jnp.int32)]
            + [pltpu.VMEM((CHUNK, dim), jnp.float32)] * NBUF
            + [pltpu.VMEM((dim, CHUNK), jnp.float32)] * NBUF
            + [pltpu.SemaphoreType.DMA] * (2 * NBUF)
        ),
        compiler_params=pltpu.CompilerParams(
            use_tc_tiling_on_sc=False, needs_layout_passes=False),
    )
    def lookup(emb_hbm, idx_hbm, out_hbm, idx_v, *bufs):
        rows_v = bufs[:NBUF]
        tbuf_v = bufs[NBUF:2 * NBUF]
        gsem = bufs[2 * NBUF:3 * NBUF]
        ssem = bufs[3 * NBUF:]
        wid = lax.axis_index("s") * nc + lax.axis_index("c")
        base = wid * chunks_per_w
        pltpu.sync_copy(idx_hbm.at[pl.ds(base, chunks_per_w)], idx_v)

        row_ids = [lax.iota(jnp.int32, LANES) + lg * LANES
                   for lg in range(CHUNK // LANES)]

        def gather(cl, b):
            return pltpu.make_async_copy(
                emb_hbm.at[idx_v.at[cl]], rows_v[b], gsem[b])

        def out_copy(cl, b, hg):
            c = base + cl
            s = c // bgroups
            bg = lax.rem(c, bgroups)
            row0 = ((s * (dim // 8) + hg) * bgroups + bg) * 8
            return pltpu.make_async_copy(
                tbuf_v[b].at[pl.ds(hg * 8, 8)], out_hbm.at[pl.ds(row0, 8)],
                ssem[b])

        iota16 = lax.iota(jnp.int32, LANES)

        def transpose(b):
            @plsc.parallel_loop(0, dim, 1, unroll=16)
            def h_body(h):
                hvec = lax.bitwise_and(iota16 + h, jnp.int32(dim - 1))
                for lg in range(CHUNK // LANES):
                    vals = plsc.load_gather(rows_v[b], [row_ids[lg], hvec])
                    plsc.store_scatter(tbuf_v[b], [hvec, row_ids[lg]], vals)

        def do_chunk(cl, b, drain_prev, fire_next):
            gather(cl, b).wait()
            if drain_prev:
                for hg in range(dim // 8):
                    out_copy(cl, b, hg).wait()
            transpose(b)
            for hg in range(dim // 8):
                out_copy(cl, b, hg).start()
            if fire_next:
                gather(cl + NBUF, b).start()

        for b in range(NBUF):
            gather(b, b).start()

        def round_body(r, carry):
            for b in range(NBUF):
                do_chunk(r * NBUF + b, b, drain_prev=True, fire_next=True)
            return carry

        for b in range(NBUF):
            do_chunk(b, b, drain_prev=False, fire_next=True)
        lax.fori_loop(1, n_rounds - 1, round_body, 0)
        c0 = (n_rounds - 1) * NBUF
        for b in range(NBUF):
            do_chunk(c0 + b, b, drain_prev=True, fire_next=False)
        for b in range(NBUF):
            for hg in range(dim // 8):
                out_copy(c0 + b, b, hg).wait()

    return lookup


def kernel(item_seqs, emb):
    bsz, seq = item_seqs.shape
    _, dim = emb.shape
    n_rows = bsz * seq
    idx2d = item_seqs.T.reshape(n_rows // CHUNK, CHUNK)
    out2 = _make_lookup(n_rows, dim, seq)(emb, idx2d)
    out5 = out2.reshape(seq, dim // 8, bsz // CHUNK, 8, CHUNK)
    return out5.transpose(2, 4, 0, 1, 3).reshape(bsz, seq, dim)
